# baseline (speedup 1.0000x reference)
import functools

import jax
import jax.numpy as jnp
from jax import lax
from jax.experimental import pallas as pl
from jax.experimental.pallas import tpu as pltpu
from jax.experimental.pallas import tpu_sc as plsc

_NC = 2
_NS = 16
_NW = _NC * _NS

_B = 16384
_D = 32
_BPW = _B // _NW
_CH = 128
_NCH = _BPW // _CH


def _sc_gather(user_emb, item_emb, uids2d, iids2d):
    mesh = plsc.VectorSubcoreMesh(core_axis_name="c", subcore_axis_name="s")

    @functools.partial(
        pl.kernel,
        out_type=(
            jax.ShapeDtypeStruct((_B, _D), jnp.float32),
            jax.ShapeDtypeStruct((_B, _D), jnp.float32),
        ),
        mesh=mesh,
        scratch_types=[
            pltpu.VMEM((_NCH, _CH), jnp.int32),
            pltpu.VMEM((_NCH, _CH), jnp.int32),
            pltpu.VMEM((_BPW, _D), jnp.float32),
            pltpu.VMEM((_BPW, _D), jnp.float32),
            pltpu.SemaphoreType.DMA,
        ],
        compiler_params=pltpu.CompilerParams(use_tc_tiling_on_sc=False),
    )
    def k(uemb_hbm, iemb_hbm, uid_hbm, iid_hbm, u_out, v_out,
          uidx_v, iidx_v, urows_v, vrows_v, sem):
        wid = lax.axis_index("s") * _NC + lax.axis_index("c")
        base = wid * _BPW
        row0 = wid * _NCH
        pltpu.sync_copy(uid_hbm.at[pl.ds(row0, _NCH)], uidx_v)
        pltpu.sync_copy(iid_hbm.at[pl.ds(row0, _NCH)], iidx_v)
        cps = []
        for j in range(_NCH):
            cps.append(pltpu.async_copy(
                uemb_hbm.at[uidx_v.at[j]], urows_v.at[pl.ds(j * _CH, _CH)], sem))
            cps.append(pltpu.async_copy(
                iemb_hbm.at[iidx_v.at[j]], vrows_v.at[pl.ds(j * _CH, _CH)], sem))
        for c in cps:
            c.wait()
        pltpu.sync_copy(urows_v, u_out.at[pl.ds(base, _BPW)])
        pltpu.sync_copy(vrows_v, v_out.at[pl.ds(base, _BPW)])

    return k(user_emb, item_emb, uids2d, iids2d)


_BLK = 2048


def _tc_mlp(u, v, w1u_t, w1v_t, b1r, w2_t, b2r, w3r, b3):
    def body(u_ref, v_ref, w1u_ref, w1v_ref, b1_ref, w2_ref, b2_ref,
             w3_ref, b3_ref, o_ref):
        h = jnp.dot(u_ref[...], w1u_ref[...], preferred_element_type=jnp.float32)
        h = h + jnp.dot(v_ref[...], w1v_ref[...], preferred_element_type=jnp.float32)
        h = jnp.maximum(h + b1_ref[...], 0.0)
        h2 = jnp.dot(h, w2_ref[...], preferred_element_type=jnp.float32)
        h2 = jnp.maximum(h2 + b2_ref[...], 0.0)
        o_ref[...] = jnp.sum(h2 * w3_ref[...], axis=1) + b3_ref[0]

    grid = (_B // _BLK,)
    const = lambda i: (0, 0)
    return pl.pallas_call(
        body,
        grid=grid,
        in_specs=[
            pl.BlockSpec((_BLK, _D), lambda i: (i, 0)),
            pl.BlockSpec((_BLK, _D), lambda i: (i, 0)),
            pl.BlockSpec((_D, _D), const),
            pl.BlockSpec((_D, _D), const),
            pl.BlockSpec((1, _D), const),
            pl.BlockSpec((_D, 16), const),
            pl.BlockSpec((1, 16), const),
            pl.BlockSpec((1, 16), const),
            pl.BlockSpec(memory_space=pltpu.SMEM),
        ],
        out_specs=pl.BlockSpec((_BLK,), lambda i: (i,)),
        out_shape=jax.ShapeDtypeStruct((_B,), jnp.float32),
    )(u, v, w1u_t, w1v_t, b1r, w2_t, b2r, w3r, b3)


def kernel(user_ids, item_ids, user_emb, item_emb, W1, b1, W2, b2, W3, b3):
    uids2d = user_ids.astype(jnp.int32).reshape(_B // _CH, _CH)
    iids2d = item_ids.astype(jnp.int32).reshape(_B // _CH, _CH)
    u, v = _sc_gather(user_emb, item_emb, uids2d, iids2d)
    w1u_t = W1[:, :_D].T
    w1v_t = W1[:, _D:].T
    out = _tc_mlp(u, v, w1u_t, w1v_t, b1.reshape(1, _D), W2.T,
                  b2.reshape(1, 16), W3, b3)
    return out


# speedup vs baseline: 2.1196x; 2.1196x over previous
import functools

import jax
import jax.numpy as jnp
from jax import lax
from jax.experimental import pallas as pl
from jax.experimental.pallas import tpu as pltpu
from jax.experimental.pallas import tpu_sc as plsc

_NC = 2
_NS = 16
_NW = _NC * _NS

_B = 16384
_D = 32
_PACK = 4
_BPW = _B // _NW
_CH = 128
_NCH = _BPW // _CH
_HALF = _BPW // 2


_Q = 262144
_QB = 2048
_NROW = 1000000


def _tc_repack(emb_t):
    def body(x0_ref, x1_ref, x2_ref, x3_ref, o_ref):
        row = lax.broadcasted_iota(jnp.int32, (_D, _PACK * _D), 0)
        col = lax.broadcasted_iota(jnp.int32, (_D, _PACK * _D), 1)
        acc = jnp.zeros((_QB, _PACK * _D), jnp.float32)
        for j, xr in enumerate((x0_ref, x1_ref, x2_ref, x3_ref)):
            ej = (col == row + _D * j).astype(jnp.float32)
            acc = acc + lax.dot_general(
                xr[...], ej, (((0,), (0,)), ((), ())),
                preferred_element_type=jnp.float32)
        o_ref[...] = acc

    grid = (_Q // _QB,)
    nspec = _Q // _QB
    last_blk = (_NROW - 1) // _QB
    in_specs = [
        pl.BlockSpec((_D, _QB), functools.partial(
            lambda j, i: (0, jnp.minimum(i + nspec * j, last_blk)), j))
        for j in range(_PACK)
    ]
    return pl.pallas_call(
        body,
        grid=grid,
        in_specs=in_specs,
        out_specs=pl.BlockSpec((_QB, _PACK * _D), lambda i: (i, 0)),
        out_shape=jax.ShapeDtypeStruct((_Q, _PACK * _D), jnp.float32),
    )(emb_t, emb_t, emb_t, emb_t)


def _sc_gather(pu, pi, uids2d, iids2d):
    mesh = plsc.VectorSubcoreMesh(core_axis_name="c", subcore_axis_name="s")

    @functools.partial(
        pl.kernel,
        out_type=(
            jax.ShapeDtypeStruct((_B, 4 * _D), jnp.float32),
            jax.ShapeDtypeStruct((_B, 4 * _D), jnp.float32),
        ),
        mesh=mesh,
        scratch_types=[
            pltpu.VMEM((_NCH, _CH), jnp.int32),
            pltpu.VMEM((_NCH, _CH), jnp.int32),
            pltpu.VMEM((_HALF, 4 * _D), jnp.float32),
            pltpu.VMEM((_HALF, 4 * _D), jnp.float32),
            pltpu.SemaphoreType.DMA,
        ],
    )
    def k(pu_hbm, pi_hbm, uid_hbm, iid_hbm, xu_out, xi_out,
          ubuf, ibuf, rbu, rbi, sem):
        wid = lax.axis_index("s") * _NC + lax.axis_index("c")
        base = wid * _BPW
        idrow = wid * _NCH
        pltpu.sync_copy(uid_hbm.at[pl.ds(idrow, _NCH)], ubuf)
        pltpu.sync_copy(iid_hbm.at[pl.ds(idrow, _NCH)], ibuf)
        for h in range(2):
            cps = []
            for j in range(2):
                cps.append(pltpu.async_copy(
                    pu_hbm.at[ubuf.at[2 * h + j]],
                    rbu.at[pl.ds(j * _CH, _CH)], sem))
                cps.append(pltpu.async_copy(
                    pi_hbm.at[ibuf.at[2 * h + j]],
                    rbi.at[pl.ds(j * _CH, _CH)], sem))
            for c in cps:
                c.wait()
            pltpu.sync_copy(rbu, xu_out.at[pl.ds(base + h * _HALF, _HALF)])
            pltpu.sync_copy(rbi, xi_out.at[pl.ds(base + h * _HALF, _HALF)])

    return k(pu, pi, uids2d, iids2d)


_BLK = 2048
_RB = _BLK // _CH


def _tc_mlp(xu3, xi3, uids2d, iids2d, w1u_t, w1v_t, b1r, w2_t, b2r, w3r, b3):
    def body(xu_ref, xi_ref, uid_ref, iid_ref, w1u_ref, w1v_ref, b1_ref,
             w2_ref, b2_ref, w3_ref, b3_ref, o_ref):
        lane_q = lax.broadcasted_iota(jnp.int32, (_RB, _CH, _PACK * _D), 2) // _D
        um = uid_ref[...] >> 18
        im = iid_ref[...] >> 18
        xum = xu_ref[...] * (lane_q == um[:, :, None]).astype(jnp.float32)
        xim = xi_ref[...] * (lane_q == im[:, :, None]).astype(jnp.float32)
        u2 = xum.reshape(_BLK, _PACK * _D)
        v2 = xim.reshape(_BLK, _PACK * _D)
        h = jnp.dot(u2, w1u_ref[...], preferred_element_type=jnp.float32)
        h = h + jnp.dot(v2, w1v_ref[...], preferred_element_type=jnp.float32)
        h = jnp.maximum(h + b1_ref[...], 0.0)
        h2 = jnp.dot(h, w2_ref[...], preferred_element_type=jnp.float32)
        h2 = jnp.maximum(h2 + b2_ref[...], 0.0)
        o_ref[...] = jnp.sum(h2 * w3_ref[...], axis=1) + b3_ref[0]

    grid = (_B // _BLK,)
    const2 = lambda i: (0, 0)
    return pl.pallas_call(
        body,
        grid=grid,
        in_specs=[
            pl.BlockSpec((_RB, _CH, 4 * _D), lambda i: (i, 0, 0)),
            pl.BlockSpec((_RB, _CH, 4 * _D), lambda i: (i, 0, 0)),
            pl.BlockSpec((_RB, _CH), lambda i: (i, 0)),
            pl.BlockSpec((_RB, _CH), lambda i: (i, 0)),
            pl.BlockSpec((_PACK * _D, _D), const2),
            pl.BlockSpec((_PACK * _D, _D), const2),
            pl.BlockSpec((1, _D), const2),
            pl.BlockSpec((_D, 16), const2),
            pl.BlockSpec((1, 16), const2),
            pl.BlockSpec((1, 16), const2),
            pl.BlockSpec(memory_space=pltpu.SMEM),
        ],
        out_specs=pl.BlockSpec((_BLK,), lambda i: (i,)),
        out_shape=jax.ShapeDtypeStruct((_B,), jnp.float32),
    )(xu3, xi3, uids2d, iids2d, w1u_t, w1v_t, b1r, w2_t, b2r, w3r, b3)


def kernel(user_ids, item_ids, user_emb, item_emb, W1, b1, W2, b2, W3, b3):
    pu = _tc_repack(user_emb.T)
    pi = _tc_repack(item_emb.T)
    uids2d = user_ids.astype(jnp.int32).reshape(_B // _CH, _CH)
    iids2d = item_ids.astype(jnp.int32).reshape(_B // _CH, _CH)
    xu, xi = _sc_gather(pu, pi, uids2d & (_Q - 1), iids2d & (_Q - 1))
    xu3 = xu.reshape(_B // _CH, _CH, 4 * _D)
    xi3 = xi.reshape(_B // _CH, _CH, 4 * _D)
    w1u_t = jnp.tile(W1[:, :_D].T, (_PACK, 1))
    w1v_t = jnp.tile(W1[:, _D:].T, (_PACK, 1))
    out = _tc_mlp(xu3, xi3, uids2d, iids2d, w1u_t, w1v_t,
                  b1.reshape(1, _D), W2.T, b2.reshape(1, 16), W3, b3)
    return out
